# baseline (speedup 1.0000x reference)
import functools

import jax
import jax.numpy as jnp
from jax.experimental import pallas as pl
from jax.experimental.pallas import tpu as pltpu


def _build_patches_t(x_ref, j, p_ref, *, cin, kh, kw, w_img, n_rows, hw):
    for ikh in range(kh):
        for ikw in range(kw):
            t = ikh * kw + ikw
            off = ikh * w_img + ikw
            m = min(n_rows, hw - off)
            p_ref[:m, t * cin:(t + 1) * cin] = (
                x_ref[j, off:off + m, :].astype(jnp.bfloat16))


def _conv_stats_kernel(w_ref, x_ref, stats_ref, p_ref, *,
                       cin, kh, kw, w_img, wo, n_rows, hw):
    _build_patches_t(x_ref, 0, p_ref, cin=cin, kh=kh, kw=kw, w_img=w_img,
                     n_rows=n_rows, hw=hw)
    y = jnp.dot(p_ref[...], w_ref[...], preferred_element_type=jnp.float32)
    row = jax.lax.broadcasted_iota(jnp.int32, (n_rows, 1), 0)
    ym = jnp.where(row % w_img < wo, y, 0.0)
    stats_ref[0, 0:1, :] = jnp.sum(ym, axis=0, keepdims=True)
    stats_ref[0, 1:2, :] = jnp.sum(ym * ym, axis=0, keepdims=True)


def _conv_norm_kernel(w_ref, stats_ref, x_ref, o_hbm, p_ref, yc_ref, sem, *,
                      cin, kh, kw, w_img, wo, ho, n_rows, hw, count, eps,
                      blk):
    i = pl.program_id(0)
    st = jnp.sum(stats_ref[...], axis=0)
    s = st[0:1, :]
    ss = st[1:2, :]
    mean = s * (1.0 / count)
    var = (ss - s * mean) * (1.0 / (count - 1.0))
    inv = 1.0 / (jnp.sqrt(var) + eps)
    _build_patches_t(x_ref, 0, p_ref, cin=cin, kh=kh, kw=kw,
                     w_img=w_img, n_rows=n_rows, hw=hw)
    half = ho // 2
    for j in range(blk):
        r0 = j * half
        rows = half if j == 0 else ho - half
        y = jnp.dot(p_ref[r0 * w_img:(r0 + rows) * w_img, :], w_ref[...],
                    preferred_element_type=jnp.float32)
        o = (y - mean) * inv
        yc_ref[j, :rows * wo, :] = jnp.concatenate(
            [o[r * w_img:r * w_img + wo, :] for r in range(rows)], axis=0)
        pltpu.make_async_copy(
            yc_ref.at[j, 0:rows * wo, :],
            o_hbm.at[r0 * wo:(r0 + rows) * wo, i, :], sem.at[j]).start()
    for j in range(blk):
        rows = half if j == 0 else ho - half
        r0 = j * half
        pltpu.make_async_copy(
            yc_ref.at[j, 0:rows * wo, :],
            o_hbm.at[r0 * wo:(r0 + rows) * wo, i, :], sem.at[j]).wait()


def kernel(X, conv_weight):
    n, cin, h, w_img = X.shape
    cout, _, kh, kw = conv_weight.shape
    ho = h - kh + 1
    wo = w_img - kw + 1
    hw = h * w_img
    n_rows = ho * w_img
    k_dim = cin * kh * kw
    count = float(n * ho * wo)
    eps = 1.0

    xt = X.transpose(0, 2, 3, 1).reshape(n, hw, cin)
    w_t = (conv_weight.transpose(2, 3, 1, 0)
           .reshape(k_dim, cout).astype(jnp.bfloat16))

    vmem_limit = 48 * 1024 * 1024

    stats = pl.pallas_call(
        functools.partial(_conv_stats_kernel, cin=cin, kh=kh, kw=kw,
                          w_img=w_img, wo=wo, n_rows=n_rows, hw=hw),
        out_shape=jax.ShapeDtypeStruct((n, 2, cout), jnp.float32),
        grid=(n,),
        in_specs=[pl.BlockSpec((k_dim, cout), lambda i: (0, 0)),
                  pl.BlockSpec((1, hw, cin), lambda i: (i, 0, 0))],
        out_specs=pl.BlockSpec((1, 2, cout), lambda i: (i, 0, 0)),
        scratch_shapes=[pltpu.VMEM((n_rows, k_dim), jnp.bfloat16)],
        compiler_params=pltpu.CompilerParams(
            dimension_semantics=("parallel",),
            vmem_limit_bytes=vmem_limit),
    )(w_t, xt)

    blk = 2
    out3 = pl.pallas_call(
        functools.partial(_conv_norm_kernel, cin=cin, kh=kh, kw=kw,
                          w_img=w_img, wo=wo, ho=ho, n_rows=n_rows, hw=hw,
                          count=count, eps=eps, blk=blk),
        out_shape=jax.ShapeDtypeStruct((ho * wo, n, cout), jnp.float32),
        grid=(n,),
        in_specs=[pl.BlockSpec((k_dim, cout), lambda i: (0, 0)),
                  pl.BlockSpec((n, 2, cout), lambda i: (0, 0, 0)),
                  pl.BlockSpec((1, hw, cin), lambda i: (i, 0, 0))],
        out_specs=pl.BlockSpec(memory_space=pl.ANY),
        scratch_shapes=[pltpu.VMEM((n_rows, k_dim), jnp.bfloat16),
                        pltpu.VMEM((blk, (ho - ho // 2) * wo, cout),
                                   jnp.float32),
                        pltpu.SemaphoreType.DMA((blk,))],
        compiler_params=pltpu.CompilerParams(
            dimension_semantics=("parallel",),
            vmem_limit_bytes=vmem_limit),
    )(w_t, stats, xt)
    return out3.reshape(ho, wo, n, cout).transpose(2, 3, 0, 1)


# speedup vs baseline: 1.1065x; 1.1065x over previous
import functools

import jax
import jax.numpy as jnp
from jax.experimental import pallas as pl
from jax.experimental.pallas import tpu as pltpu


def _build_patches_t(x_ref, j, p_ref, *, cin, kh, kw, w_img, n_rows, hw):
    for ikh in range(kh):
        for ikw in range(kw):
            t = ikh * kw + ikw
            off = ikh * w_img + ikw
            m = min(n_rows, hw - off)
            p_ref[:m, t * cin:(t + 1) * cin] = (
                x_ref[j, off:off + m, :].astype(jnp.bfloat16))


def _conv_stats_kernel(w_ref, x_ref, stats_ref, p_ref, *,
                       cin, kh, kw, w_img, wo, n_rows, hw):
    _build_patches_t(x_ref, 0, p_ref, cin=cin, kh=kh, kw=kw, w_img=w_img,
                     n_rows=n_rows, hw=hw)
    y = jnp.dot(p_ref[...], w_ref[...], preferred_element_type=jnp.float32)
    row = jax.lax.broadcasted_iota(jnp.int32, (n_rows, 1), 0)
    ym = jnp.where(row % w_img < wo, y, 0.0)
    stats_ref[0, 0:1, :] = jnp.sum(ym, axis=0, keepdims=True)
    stats_ref[0, 1:2, :] = jnp.sum(ym * ym, axis=0, keepdims=True)


def _conv_norm_kernel(w_ref, stats_ref, x_ref, o_hbm, p_ref, yc_ref, sem, *,
                      cin, kh, kw, w_img, wo, ho, n_rows, hw, count, eps):
    i = pl.program_id(0)
    _build_patches_t(x_ref, 0, p_ref, cin=cin, kh=kh, kw=kw, w_img=w_img,
                     n_rows=n_rows, hw=hw)
    y = jnp.dot(p_ref[...], w_ref[...], preferred_element_type=jnp.float32)
    st = jnp.sum(stats_ref[...], axis=0)
    s = st[0:1, :]
    ss = st[1:2, :]
    mean = s * (1.0 / count)
    var = (ss - s * mean) * (1.0 / (count - 1.0))
    inv = 1.0 / (jnp.sqrt(var) + eps)
    o = (y - mean) * inv
    yc_ref[...] = jnp.concatenate(
        [o[r * w_img:r * w_img + wo, :] for r in range(ho)], axis=0)
    cp = pltpu.make_async_copy(yc_ref, o_hbm.at[:, i, :], sem)
    cp.start()
    cp.wait()


def kernel(X, conv_weight):
    n, cin, h, w_img = X.shape
    cout, _, kh, kw = conv_weight.shape
    ho = h - kh + 1
    wo = w_img - kw + 1
    hw = h * w_img
    n_rows = ho * w_img
    k_dim = cin * kh * kw
    count = float(n * ho * wo)
    eps = 1.0

    xt = X.transpose(0, 2, 3, 1).reshape(n, hw, cin)
    w_t = (conv_weight.transpose(2, 3, 1, 0)
           .reshape(k_dim, cout).astype(jnp.bfloat16))

    vmem_limit = 48 * 1024 * 1024

    stats = pl.pallas_call(
        functools.partial(_conv_stats_kernel, cin=cin, kh=kh, kw=kw,
                          w_img=w_img, wo=wo, n_rows=n_rows, hw=hw),
        out_shape=jax.ShapeDtypeStruct((n, 2, cout), jnp.float32),
        grid=(n,),
        in_specs=[pl.BlockSpec((k_dim, cout), lambda i: (0, 0)),
                  pl.BlockSpec((1, hw, cin), lambda i: (i, 0, 0))],
        out_specs=pl.BlockSpec((1, 2, cout), lambda i: (i, 0, 0)),
        scratch_shapes=[pltpu.VMEM((n_rows, k_dim), jnp.bfloat16)],
        compiler_params=pltpu.CompilerParams(
            dimension_semantics=("parallel",),
            vmem_limit_bytes=vmem_limit),
    )(w_t, xt)

    out3 = pl.pallas_call(
        functools.partial(_conv_norm_kernel, cin=cin, kh=kh, kw=kw,
                          w_img=w_img, wo=wo, ho=ho, n_rows=n_rows, hw=hw,
                          count=count, eps=eps),
        out_shape=jax.ShapeDtypeStruct((ho * wo, n, cout), jnp.float32),
        grid=(n,),
        in_specs=[pl.BlockSpec((k_dim, cout), lambda i: (0, 0)),
                  pl.BlockSpec((n, 2, cout), lambda i: (0, 0, 0)),
                  pl.BlockSpec((1, hw, cin), lambda i: (i, 0, 0))],
        out_specs=pl.BlockSpec(memory_space=pl.ANY),
        scratch_shapes=[pltpu.VMEM((n_rows, k_dim), jnp.bfloat16),
                        pltpu.VMEM((ho * wo, cout), jnp.float32),
                        pltpu.SemaphoreType.DMA],
        compiler_params=pltpu.CompilerParams(
            dimension_semantics=("parallel",),
            vmem_limit_bytes=vmem_limit),
    )(w_t, stats, xt)
    return out3.reshape(ho, wo, n, cout).transpose(2, 3, 0, 1)
